# baseline (speedup 1.0000x reference)
import functools

import jax
import jax.numpy as jnp
import numpy as np
from jax import lax
from jax.experimental import pallas as pl
from jax.experimental.pallas import tpu as pltpu
from jax.experimental.pallas import tpu_sc as plsc

_L = 8
_T = 1024
_B_G = float(np.exp(np.log(20 * 0.5 / 2) / (_L - 1)))
_RES = [int(np.floor(2 * _B_G**l)) for l in range(_L)]
_C1 = np.int32(np.uint32(2654435761).astype(np.int32))
_C2 = np.int32(805459861)

_NW = 32
_CNK = 2048


def _sc_encode(xf, grid_flat):
    n = xf.shape[0] // 3
    npts = n // _NW
    nchunk = npts // _CNK
    mesh = plsc.VectorSubcoreMesh(core_axis_name="c", subcore_axis_name="s")

    @functools.partial(
        pl.kernel,
        out_type=jax.ShapeDtypeStruct((16, n), jnp.float32),
        mesh=mesh,
        scratch_types=[
            pltpu.VMEM((_L * _T * 2,), jnp.float32),
            pltpu.VMEM((3 * _CNK,), jnp.float32),
            pltpu.VMEM((16, _CNK), jnp.float32),
        ],
        compiler_params=pltpu.CompilerParams(needs_layout_passes=False),
    )
    def enc(xf_hbm, grid_hbm, ft_hbm, gv, xv, fv):
        wid = lax.axis_index("s") * 2 + lax.axis_index("c")
        pltpu.sync_copy(grid_hbm, gv)
        base0 = wid * npts
        i3 = lax.iota(jnp.int32, 16) * 3

        def chunk_body(ci, _):
            base = base0 + ci * _CNK
            pltpu.sync_copy(xf_hbm.at[pl.ds(base * 3, _CNK * 3)], xv)

            def pt_body(i, _):
                sl = pl.ds(i * 16, 16)
                ofs = i3 + i * 48
                xs = plsc.load_gather(xv, [ofs])
                ys = plsc.load_gather(xv, [ofs + 1])
                zs = plsc.load_gather(xv, [ofs + 2])
                for l in range(_L):
                    res = float(_RES[l])
                    px = xs * res
                    py = ys * res
                    pz = zs * res
                    ix = px.astype(jnp.int32)
                    iy = py.astype(jnp.int32)
                    iz = pz.astype(jnp.int32)
                    wx1 = px - ix.astype(jnp.float32)
                    wy1 = py - iy.astype(jnp.float32)
                    wz1 = pz - iz.astype(jnp.float32)
                    hy = (iy * _C1, (iy + 1) * _C1)
                    hz = (iz * _C2, (iz + 1) * _C2)
                    hx = (ix, ix + 1)
                    wxs = (1.0 - wx1, wx1)
                    wys = (1.0 - wy1, wy1)
                    wzs = (1.0 - wz1, wz1)
                    acc0 = None
                    acc1 = None
                    for dx in (0, 1):
                        for dy in (0, 1):
                            hxy = hx[dx] ^ hy[dy]
                            wxy = wxs[dx] * wys[dy]
                            for dz in (0, 1):
                                h = (hxy ^ hz[dz]) & (_T - 1)
                                idx = h * 2 + (l * _T * 2)
                                g0 = plsc.load_gather(gv, [idx])
                                g1 = plsc.load_gather(gv, [idx + 1])
                                w = wxy * wzs[dz]
                                if acc0 is None:
                                    acc0 = w * g0
                                    acc1 = w * g1
                                else:
                                    acc0 = acc0 + w * g0
                                    acc1 = acc1 + w * g1
                    fv[2 * l, sl] = acc0
                    fv[2 * l + 1, sl] = acc1
                return 0

            lax.fori_loop(0, _CNK // 16, pt_body, 0)
            pltpu.sync_copy(fv, ft_hbm.at[:, pl.ds(base, _CNK)])
            return 0

        lax.fori_loop(0, nchunk, chunk_body, 0)

    return enc(xf, grid_flat)


def _mlp_body(ft_ref, w1_ref, w2_ref, r1_ref, r2_ref, r3_ref, o_ref):
    f32 = jnp.float32
    ft = ft_ref[...]
    t = lax.dot_general(ft, w1_ref[...], (((0,), (0,)), ((), ())),
                        preferred_element_type=f32)
    t = jnp.maximum(t, 0.0)
    w21 = jnp.dot(w2_ref[...], r1_ref[...], preferred_element_type=f32)
    r = jnp.maximum(jnp.dot(t, w21, preferred_element_type=f32), 0.0)
    r = jnp.maximum(jnp.dot(r, r2_ref[...], preferred_element_type=f32), 0.0)
    o_ref[...] = jax.nn.sigmoid(
        jnp.dot(r, r3_ref[...], preferred_element_type=f32))


@functools.partial(jax.jit, static_argnames=("nb",))
def _run(x, grid, W1, W2, R1, R2, R3, nb=8192):
    n = x.shape[0]
    ft = _sc_encode(x.reshape(-1), grid.reshape(-1))
    full = lambda a: pl.BlockSpec(a.shape, lambda i: (0,) * a.ndim)
    out = pl.pallas_call(
        _mlp_body,
        grid=(n // nb,),
        in_specs=[
            pl.BlockSpec((16, nb), lambda i: (0, i)),
            full(W1), full(W2), full(R1), full(R2), full(R3),
        ],
        out_specs=pl.BlockSpec((nb, 1), lambda i: (i, 0)),
        out_shape=jax.ShapeDtypeStruct((n, 1), jnp.float32),
    )(ft, W1, W2, R1, R2, R3)
    return out


def kernel(x, grid, W1, W2, R1, R2, R3):
    return _run(x, grid, W1, W2, R1, R2, R3)


# speedup vs baseline: 1.1662x; 1.1662x over previous
import functools

import jax
import jax.numpy as jnp
import numpy as np
from jax import lax
from jax.experimental import pallas as pl
from jax.experimental.pallas import tpu as pltpu
from jax.experimental.pallas import tpu_sc as plsc

_L = 8
_T = 1024
_B_G = float(np.exp(np.log(20 * 0.5 / 2) / (_L - 1)))
_RES = [int(np.floor(2 * _B_G**l)) for l in range(_L)]
_C1 = np.int32(np.uint32(2654435761).astype(np.int32))
_C2 = np.int32(805459861)

_NW = 32
_CNK = 4096
_C1_2 = np.int32(np.uint32(2 * 2654435761 % (2**32)).astype(np.int32))
_C2_2 = np.int32(np.uint32(2 * 805459861 % (2**32)).astype(np.int32))


def _sc_encode(xf, grid_flat):
    n = xf.shape[0] // 3
    npts = n // _NW
    nchunk = npts // _CNK
    mesh = plsc.VectorSubcoreMesh(core_axis_name="c", subcore_axis_name="s")

    @functools.partial(
        pl.kernel,
        out_type=jax.ShapeDtypeStruct((16, n), jnp.float32),
        mesh=mesh,
        scratch_types=[
            pltpu.VMEM((_L * _T * 2,), jnp.float32),
            pltpu.VMEM((3 * _CNK,), jnp.float32),
            pltpu.VMEM((16, _CNK), jnp.float32),
        ],
        compiler_params=pltpu.CompilerParams(needs_layout_passes=False),
    )
    def enc(xf_hbm, grid_hbm, ft_hbm, gv, xv, fv):
        wid = lax.axis_index("s") * 2 + lax.axis_index("c")
        pltpu.sync_copy(grid_hbm, gv)
        base0 = wid * npts
        i3 = lax.iota(jnp.int32, 16) * 3

        def chunk_body(ci, _):
            base = base0 + ci * _CNK
            pltpu.sync_copy(xf_hbm.at[pl.ds(base * 3, _CNK * 3)], xv)

            @functools.partial(plsc.parallel_loop, 0, _CNK // 16, unroll=2)
            def pt_body(i):
                sl = pl.ds(i * 16, 16)
                ofs = i3 + i * 48
                xs = plsc.load_gather(xv, [ofs])
                ys = plsc.load_gather(xv, [ofs + 1])
                zs = plsc.load_gather(xv, [ofs + 2])
                for l in range(_L):
                    res = float(_RES[l])
                    loff = l * _T * 2
                    px = xs * res
                    py = ys * res
                    pz = zs * res
                    ix = px.astype(jnp.int32)
                    iy = py.astype(jnp.int32)
                    iz = pz.astype(jnp.int32)
                    wx1 = px - ix.astype(jnp.float32)
                    wy1 = py - iy.astype(jnp.float32)
                    wz1 = pz - iz.astype(jnp.float32)
                    ix2 = ix * 2
                    hy20 = iy * _C1_2
                    hz20 = iz * _C2_2
                    hx = (ix2, ix2 + 2)
                    hy = (hy20, hy20 + _C1_2)
                    hz = (hz20, hz20 + _C2_2)
                    wxs = (1.0 - wx1, wx1)
                    wys = (1.0 - wy1, wy1)
                    wzs = (1.0 - wz1, wz1)
                    acc0 = None
                    acc1 = None
                    for dx in (0, 1):
                        for dy in (0, 1):
                            hxy = hx[dx] ^ hy[dy]
                            wxy = wxs[dx] * wys[dy]
                            for dz in (0, 1):
                                idx = ((hxy ^ hz[dz]) & (2 * _T - 1)) | loff
                                g0 = plsc.load_gather(gv, [idx])
                                g1 = plsc.load_gather(gv, [idx + 1])
                                w = wxy * wzs[dz]
                                if acc0 is None:
                                    acc0 = w * g0
                                    acc1 = w * g1
                                else:
                                    acc0 = acc0 + w * g0
                                    acc1 = acc1 + w * g1
                    fv[2 * l, sl] = acc0
                    fv[2 * l + 1, sl] = acc1
            pltpu.sync_copy(fv, ft_hbm.at[:, pl.ds(base, _CNK)])
            return 0

        lax.fori_loop(0, nchunk, chunk_body, 0)

    return enc(xf, grid_flat)


def _mlp_body(ft_ref, w1_ref, w2_ref, r1_ref, r2_ref, r3_ref, o_ref):
    f32 = jnp.float32
    ft = ft_ref[...]
    t = lax.dot_general(ft, w1_ref[...], (((0,), (0,)), ((), ())),
                        preferred_element_type=f32)
    t = jnp.maximum(t, 0.0)
    w21 = jnp.dot(w2_ref[...], r1_ref[...], preferred_element_type=f32)
    r = jnp.maximum(jnp.dot(t, w21, preferred_element_type=f32), 0.0)
    r = jnp.maximum(jnp.dot(r, r2_ref[...], preferred_element_type=f32), 0.0)
    o_ref[...] = jax.nn.sigmoid(
        jnp.dot(r, r3_ref[...], preferred_element_type=f32))


@functools.partial(jax.jit, static_argnames=("nb",))
def _run(x, grid, W1, W2, R1, R2, R3, nb=2048):
    n = x.shape[0]
    ft = _sc_encode(x.reshape(-1), grid.reshape(-1))
    full = lambda a: pl.BlockSpec(a.shape, lambda i: (0,) * a.ndim)
    out = pl.pallas_call(
        _mlp_body,
        grid=(n // nb,),
        in_specs=[
            pl.BlockSpec((16, nb), lambda i: (0, i)),
            full(W1), full(W2), full(R1), full(R2), full(R3),
        ],
        out_specs=pl.BlockSpec((nb, 1), lambda i: (i, 0)),
        out_shape=jax.ShapeDtypeStruct((n, 1), jnp.float32),
    )(ft, W1, W2, R1, R2, R3)
    return out


def kernel(x, grid, W1, W2, R1, R2, R3):
    return _run(x, grid, W1, W2, R1, R2, R3)


# speedup vs baseline: 1.3548x; 1.1617x over previous
import functools

import jax
import jax.numpy as jnp
import numpy as np
from jax import lax
from jax.experimental import pallas as pl
from jax.experimental.pallas import tpu as pltpu
from jax.experimental.pallas import tpu_sc as plsc

_L = 8
_T = 1024
_B_G = float(np.exp(np.log(20 * 0.5 / 2) / (_L - 1)))
_RES = [int(np.floor(2 * _B_G**l)) for l in range(_L)]
_C1 = np.int32(np.uint32(2654435761).astype(np.int32))
_C2 = np.int32(805459861)

_NW = 32
_CNK = 4096
_C1_2 = np.int32(np.uint32(2 * 2654435761 % (2**32)).astype(np.int32))
_C2_2 = np.int32(np.uint32(2 * 805459861 % (2**32)).astype(np.int32))


def _sc_encode(xf, grid_flat):
    n = xf.shape[0] // 3
    npts = n // _NW
    nchunk = npts // _CNK
    mesh = plsc.VectorSubcoreMesh(core_axis_name="c", subcore_axis_name="s")

    @functools.partial(
        pl.kernel,
        out_type=jax.ShapeDtypeStruct((16, n), jnp.float32),
        mesh=mesh,
        scratch_types=[
            pltpu.VMEM((_L * _T * 2,), jnp.float32),
            pltpu.VMEM((3 * _CNK,), jnp.float32),
            pltpu.VMEM((16, _CNK), jnp.float32),
        ],
        compiler_params=pltpu.CompilerParams(needs_layout_passes=False),
    )
    def enc(xf_hbm, grid_hbm, ft_hbm, gv, xv, fv):
        wid = lax.axis_index("s") * 2 + lax.axis_index("c")
        pltpu.sync_copy(grid_hbm, gv)
        base0 = wid * npts
        i3 = lax.iota(jnp.int32, 16) * 3

        def chunk_body(ci, _):
            base = base0 + ci * _CNK
            pltpu.sync_copy(xf_hbm.at[pl.ds(base * 3, _CNK * 3)], xv)

            def pt_body(i, _):
                sl = pl.ds(i * 16, 16)
                ofs = i3 + i * 48
                xs = plsc.load_gather(xv, [ofs])
                ys = plsc.load_gather(xv, [ofs + 1])
                zs = plsc.load_gather(xv, [ofs + 2])
                for l in range(_L):
                    res = float(_RES[l])
                    loff = l * _T * 2
                    px = xs * res
                    py = ys * res
                    pz = zs * res
                    ix = px.astype(jnp.int32)
                    iy = py.astype(jnp.int32)
                    iz = pz.astype(jnp.int32)
                    wx1 = px - ix.astype(jnp.float32)
                    wy1 = py - iy.astype(jnp.float32)
                    wz1 = pz - iz.astype(jnp.float32)
                    ix2 = ix * 2
                    hy20 = iy * _C1_2
                    hz20 = iz * _C2_2
                    hx = (ix2, ix2 + 2)
                    hy = (hy20, hy20 + _C1_2)
                    hz = (hz20, hz20 + _C2_2)
                    wxs = (1.0 - wx1, wx1)
                    wys = (1.0 - wy1, wy1)
                    wzs = (1.0 - wz1, wz1)
                    acc0 = None
                    acc1 = None
                    for dx in (0, 1):
                        for dy in (0, 1):
                            hxy = hx[dx] ^ hy[dy]
                            wxy = wxs[dx] * wys[dy]
                            for dz in (0, 1):
                                idx = ((hxy ^ hz[dz]) & (2 * _T - 1)) | loff
                                g0 = plsc.load_gather(gv, [idx])
                                g1 = plsc.load_gather(gv, [idx + 1])
                                w = wxy * wzs[dz]
                                if acc0 is None:
                                    acc0 = w * g0
                                    acc1 = w * g1
                                else:
                                    acc0 = acc0 + w * g0
                                    acc1 = acc1 + w * g1
                    fv[2 * l, sl] = acc0
                    fv[2 * l + 1, sl] = acc1
                return 0

            lax.fori_loop(0, _CNK // 16, pt_body, 0)
            pltpu.sync_copy(fv, ft_hbm.at[:, pl.ds(base, _CNK)])
            return 0

        lax.fori_loop(0, nchunk, chunk_body, 0)

    return enc(xf, grid_flat)


def _mlp_body(ft_ref, w1_ref, w2_ref, r1_ref, r2_ref, r3_ref, o_ref):
    f32 = jnp.float32
    cn = (((0,), (0,)), ((), ()))
    ft = ft_ref[...]
    t = lax.dot_general(w1_ref[...], ft, cn,
                        preferred_element_type=f32)
    t = jnp.maximum(t, 0.0)
    w21 = jnp.dot(w2_ref[...], r1_ref[...], preferred_element_type=f32)
    r = jnp.maximum(lax.dot_general(w21, t, cn,
                                    preferred_element_type=f32), 0.0)
    r = jnp.maximum(lax.dot_general(r2_ref[...], r, cn,
                                    preferred_element_type=f32), 0.0)
    o_ref[...] = jax.nn.sigmoid(
        lax.dot_general(r3_ref[...], r, cn, preferred_element_type=f32))


@functools.partial(jax.jit, static_argnames=("nb",))
def _run(x, grid, W1, W2, R1, R2, R3, nb=16384):
    n = x.shape[0]
    ft = _sc_encode(x.reshape(-1), grid.reshape(-1))
    full = lambda a: pl.BlockSpec(a.shape, lambda i: (0,) * a.ndim)
    out = pl.pallas_call(
        _mlp_body,
        grid=(n // nb,),
        in_specs=[
            pl.BlockSpec((16, nb), lambda i: (0, i)),
            full(W1), full(W2), full(R1), full(R2), full(R3),
        ],
        out_specs=pl.BlockSpec((1, nb), lambda i: (0, i)),
        out_shape=jax.ShapeDtypeStruct((1, n), jnp.float32),
    )(ft, W1, W2, R1, R2, R3)
    return out.reshape(n, 1)


def kernel(x, grid, W1, W2, R1, R2, R3):
    return _run(x, grid, W1, W2, R1, R2, R3)


# speedup vs baseline: 2.4601x; 1.8159x over previous
import functools

import jax
import jax.numpy as jnp
import numpy as np
from jax import lax
from jax.experimental import pallas as pl
from jax.experimental.pallas import tpu as pltpu
from jax.experimental.pallas import tpu_sc as plsc

_L = 8
_T = 1024
_B_G = float(np.exp(np.log(20 * 0.5 / 2) / (_L - 1)))
_RES = [int(np.floor(2 * _B_G**l)) for l in range(_L)]
_C1 = np.int32(np.uint32(2654435761).astype(np.int32))
_C2 = np.int32(805459861)

_NW = 32
_CNK = 4096
_C1_2 = np.int32(np.uint32(2 * 2654435761 % (2**32)).astype(np.int32))
_C2_2 = np.int32(np.uint32(2 * 805459861 % (2**32)).astype(np.int32))


def _sc_encode(xs_a, ys_a, zs_a, grid_flat):
    n = xs_a.shape[0]
    npts = n // _NW
    nchunk = npts // _CNK
    mesh = plsc.VectorSubcoreMesh(core_axis_name="c", subcore_axis_name="s")

    @functools.partial(
        pl.kernel,
        out_type=jax.ShapeDtypeStruct((16, n), jnp.float32),
        mesh=mesh,
        scratch_types=[
            pltpu.VMEM((_L * _T * 2,), jnp.float32),
            pltpu.VMEM((_CNK,), jnp.float32),
            pltpu.VMEM((_CNK,), jnp.float32),
            pltpu.VMEM((_CNK,), jnp.float32),
            pltpu.VMEM((16, _CNK), jnp.float32),
        ],
        compiler_params=pltpu.CompilerParams(needs_layout_passes=False),
    )
    def enc(xs_hbm, ys_hbm, zs_hbm, grid_hbm, ft_hbm, gv, xv, yv, zv, fv):
        wid = lax.axis_index("s") * 2 + lax.axis_index("c")
        pltpu.sync_copy(grid_hbm, gv)
        base0 = wid * npts

        def chunk_body(ci, _):
            base = base0 + ci * _CNK
            pltpu.sync_copy(xs_hbm.at[pl.ds(base, _CNK)], xv)
            pltpu.sync_copy(ys_hbm.at[pl.ds(base, _CNK)], yv)
            pltpu.sync_copy(zs_hbm.at[pl.ds(base, _CNK)], zv)

            def pt_body(i, _):
                sl = pl.ds(i * 16, 16)
                xs = xv[sl]
                ys = yv[sl]
                zs = zv[sl]
                for l in range(_L):
                    res = float(_RES[l])
                    loff = l * _T * 2
                    px = xs * res
                    py = ys * res
                    pz = zs * res
                    ix = px.astype(jnp.int32)
                    iy = py.astype(jnp.int32)
                    iz = pz.astype(jnp.int32)
                    wx1 = px - ix.astype(jnp.float32)
                    wy1 = py - iy.astype(jnp.float32)
                    wz1 = pz - iz.astype(jnp.float32)
                    ix2 = ix * 2
                    hy20 = iy * _C1_2
                    hz20 = iz * _C2_2
                    hx = (ix2, ix2 + 2)
                    hy = (hy20, hy20 + _C1_2)
                    hz = (hz20, hz20 + _C2_2)
                    wxs = (1.0 - wx1, wx1)
                    wys = (1.0 - wy1, wy1)
                    wzs = (1.0 - wz1, wz1)
                    acc0 = None
                    acc1 = None
                    for dx in (0, 1):
                        for dy in (0, 1):
                            hxy = hx[dx] ^ hy[dy]
                            wxy = wxs[dx] * wys[dy]
                            for dz in (0, 1):
                                idx = ((hxy ^ hz[dz]) & (2 * _T - 1)) | loff
                                g0 = plsc.load_gather(gv, [idx])
                                g1 = plsc.load_gather(gv, [idx + 1])
                                w = wxy * wzs[dz]
                                if acc0 is None:
                                    acc0 = w * g0
                                    acc1 = w * g1
                                else:
                                    acc0 = acc0 + w * g0
                                    acc1 = acc1 + w * g1
                    fv[2 * l, sl] = acc0
                    fv[2 * l + 1, sl] = acc1
                return 0

            lax.fori_loop(0, _CNK // 16, pt_body, 0)
            pltpu.sync_copy(fv, ft_hbm.at[:, pl.ds(base, _CNK)])
            return 0

        lax.fori_loop(0, nchunk, chunk_body, 0)

    return enc(xs_a, ys_a, zs_a, grid_flat)


def _mlp_body(ft_ref, w1_ref, w2_ref, r1_ref, r2_ref, r3_ref, o_ref):
    f32 = jnp.float32
    cn = (((0,), (0,)), ((), ()))
    ft = ft_ref[...]
    t = lax.dot_general(w1_ref[...], ft, cn,
                        preferred_element_type=f32)
    t = jnp.maximum(t, 0.0)
    w21 = jnp.dot(w2_ref[...], r1_ref[...], preferred_element_type=f32)
    r = jnp.maximum(lax.dot_general(w21, t, cn,
                                    preferred_element_type=f32), 0.0)
    r = jnp.maximum(lax.dot_general(r2_ref[...], r, cn,
                                    preferred_element_type=f32), 0.0)
    o_ref[...] = jax.nn.sigmoid(
        lax.dot_general(r3_ref[...], r, cn, preferred_element_type=f32))


@functools.partial(jax.jit, static_argnames=("nb",))
def _run(x, grid, W1, W2, R1, R2, R3, nb=16384):
    n = x.shape[0]
    ft = _sc_encode(x[:, 0], x[:, 1], x[:, 2], grid.reshape(-1))
    full = lambda a: pl.BlockSpec(a.shape, lambda i: (0,) * a.ndim)
    out = pl.pallas_call(
        _mlp_body,
        grid=(n // nb,),
        in_specs=[
            pl.BlockSpec((16, nb), lambda i: (0, i)),
            full(W1), full(W2), full(R1), full(R2), full(R3),
        ],
        out_specs=pl.BlockSpec((1, nb), lambda i: (0, i)),
        out_shape=jax.ShapeDtypeStruct((1, n), jnp.float32),
    )(ft, W1, W2, R1, R2, R3)
    return out.reshape(n, 1)


def kernel(x, grid, W1, W2, R1, R2, R3):
    return _run(x, grid, W1, W2, R1, R2, R3)
